# baseline (speedup 1.0000x reference)
import functools

import jax
import jax.numpy as jnp
from jax import lax
from jax.experimental import pallas as pl
from jax.experimental.pallas import tpu as pltpu
from jax.experimental.pallas import tpu_sc as plsc

BATCH = 64
SEQ = 512
EMBD = 512
NW = 32
PW = SEQ // NW
CB = 4
NCHUNK = BATCH // CB
LANES = 16
VECS = EMBD // LANES


def _emb_body(x_hbm, tok_hbm, pos_hbm, out_hbm,
              pos_v, idx0, idx1, rows0, rows1, gsem0, gsem1, ssem0, ssem1):
    wid = lax.axis_index("s") * 2 + lax.axis_index("c")
    p0 = wid * PW
    pltpu.sync_copy(pos_hbm.at[pl.ds(p0, PW)], pos_v)

    idx = (idx0, idx1)
    rows = (rows0, rows1)
    gsem = (gsem0, gsem1)
    ssem = (ssem0, ssem1)

    def load_idx(cc, buf):
        for bb in range(CB):
            pltpu.sync_copy(x_hbm.at[cc * CB + bb, pl.ds(p0, PW)],
                            idx[buf].at[pl.ds(bb * PW, PW)])

    def start_gather(cc, buf):
        load_idx(cc, buf)
        return pltpu.async_copy(tok_hbm.at[idx[buf]], rows[buf], gsem[buf])

    def add_pos(buf):
        r = rows[buf]

        def body(row, carry):
            j = lax.rem(row, PW)
            for k in range(VECS):
                sl = pl.ds(k * LANES, LANES)
                r[row, sl] = r[row, sl] + pos_v[j, sl]
            return carry

        lax.fori_loop(0, CB * PW, body, 0)

    def start_scatter(cc, buf):
        hs = []
        for bb in range(CB):
            b = cc * CB + bb
            hs.append(pltpu.async_copy(
                rows[buf].at[pl.ds(bb * PW, PW)],
                out_hbm.at[pl.ds(b * SEQ + p0, PW)],
                ssem[buf]))
        return hs

    g = [None] * NCHUNK
    s = [None] * NCHUNK
    g[0] = start_gather(0, 0)
    for cc in range(NCHUNK):
        buf = cc % 2
        if cc + 1 < NCHUNK:
            if cc >= 1:
                for h in s[cc - 1]:
                    h.wait()
            g[cc + 1] = start_gather(cc + 1, 1 - buf)
        g[cc].wait()
        add_pos(buf)
        s[cc] = start_scatter(cc, buf)
    for h in s[NCHUNK - 2]:
        h.wait()
    for h in s[NCHUNK - 1]:
        h.wait()


def kernel(x, token_table, position_table):
    xi = x.astype(jnp.int32)
    mesh = plsc.VectorSubcoreMesh(core_axis_name="c", subcore_axis_name="s")
    f = functools.partial(
        pl.kernel,
        mesh=mesh,
        out_type=jax.ShapeDtypeStruct((BATCH * SEQ, EMBD), jnp.float32),
        scratch_types=[
            pltpu.VMEM((PW, EMBD), jnp.float32),
            pltpu.VMEM((CB * PW,), jnp.int32),
            pltpu.VMEM((CB * PW,), jnp.int32),
            pltpu.VMEM((CB * PW, EMBD), jnp.float32),
            pltpu.VMEM((CB * PW, EMBD), jnp.float32),
            pltpu.SemaphoreType.DMA,
            pltpu.SemaphoreType.DMA,
            pltpu.SemaphoreType.DMA,
            pltpu.SemaphoreType.DMA,
        ],
    )(_emb_body)
    out = f(xi, token_table, position_table)
    return out.reshape(BATCH, SEQ, EMBD)


# speedup vs baseline: 2.8533x; 2.8533x over previous
import functools

import jax
import jax.numpy as jnp
from jax import lax
from jax.experimental import pallas as pl
from jax.experimental.pallas import tpu as pltpu
from jax.experimental.pallas import tpu_sc as plsc

BATCH = 64
SEQ = 512
EMBD = 512
NW = 32
PW = SEQ // NW
LANES = 16
VECS = EMBD // LANES
BBLK = BATCH // LANES


def _emb_body(x_hbm, tok_hbm, pos_hbm, out_hbm,
              pos_v, tokid_v, oidx2_v, oidxf_v,
              rows0, rows1, gsem0, gsem1, ssem0, ssem1):
    wid = lax.axis_index("s") * 2 + lax.axis_index("c")
    p0 = wid * PW
    pltpu.sync_copy(pos_hbm.at[pl.ds(p0, PW)], pos_v)

    bvec = lax.iota(jnp.int32, LANES) * SEQ
    for j in range(PW):
        for kk in range(BBLK):
            val = bvec + (kk * LANES * SEQ + p0 + j)
            sl = pl.ds(kk * LANES, LANES)
            oidx2_v[j, sl] = val
            oidxf_v[pl.ds(j * BATCH + kk * LANES, LANES)] = val
    pltpu.sync_copy(x_hbm.at[oidxf_v], tokid_v)

    rows = (rows0, rows1)
    gsem = (gsem0, gsem1)
    ssem = (ssem0, ssem1)

    def start_gather(j, buf):
        return pltpu.async_copy(
            tok_hbm.at[tokid_v.at[pl.ds(j * BATCH, BATCH)]], rows[buf],
            gsem[buf])

    def add_pos(j, buf):
        r = rows[buf]
        pv = [pos_v[j, pl.ds(k * LANES, LANES)] for k in range(VECS)]

        def body(row, carry):
            for k in range(VECS):
                sl = pl.ds(k * LANES, LANES)
                r[row, sl] = r[row, sl] + pv[k]
            return carry

        lax.fori_loop(0, BATCH, body, 0)

    def start_scatter(j, buf):
        return pltpu.async_copy(rows[buf], out_hbm.at[oidx2_v.at[j]],
                                ssem[buf])

    g = [None] * PW
    s = [None] * PW
    g[0] = start_gather(0, 0)
    for j in range(PW):
        buf = j % 2
        if j + 1 < PW:
            if j >= 1:
                s[j - 1].wait()
            g[j + 1] = start_gather(j + 1, 1 - buf)
        g[j].wait()
        add_pos(j, buf)
        s[j] = start_scatter(j, buf)
    s[PW - 2].wait()
    s[PW - 1].wait()


def kernel(x, token_table, position_table):
    xf = x.reshape(-1).astype(jnp.int32)
    mesh = plsc.VectorSubcoreMesh(core_axis_name="c", subcore_axis_name="s")
    f = functools.partial(
        pl.kernel,
        mesh=mesh,
        out_type=jax.ShapeDtypeStruct((BATCH * SEQ, EMBD), jnp.float32),
        scratch_types=[
            pltpu.VMEM((PW, EMBD), jnp.float32),
            pltpu.VMEM((PW * BATCH,), jnp.int32),
            pltpu.VMEM((PW, BATCH), jnp.int32),
            pltpu.VMEM((PW * BATCH,), jnp.int32),
            pltpu.VMEM((BATCH, EMBD), jnp.float32),
            pltpu.VMEM((BATCH, EMBD), jnp.float32),
            pltpu.SemaphoreType.DMA,
            pltpu.SemaphoreType.DMA,
            pltpu.SemaphoreType.DMA,
            pltpu.SemaphoreType.DMA,
        ],
    )(_emb_body)
    out = f(xf, token_table, position_table)
    return out.reshape(BATCH, SEQ, EMBD)
